# Initial kernel scaffold; baseline (speedup 1.0000x reference)
import jax
import jax.numpy as jnp
from jax.experimental import pallas as pl


def kernel(t, x, edge_index, edge_weight):
    raise NotImplementedError("write your pallas kernel here")


# speedup vs baseline: 70.9569x; 70.9569x over previous
import functools

import jax
import jax.numpy as jnp
from jax import lax
from jax.experimental import pallas as pl
from jax.experimental.pallas import tpu as pltpu
from jax.experimental.pallas import tpu_sc as plsc

N_NODES = 100000
N_EDGES = 1600000
U = 3.5
D = 2.0

LANES = 16
NC = 2
NS = 16
NW = NC * NS

N_PAD = 100352
SLICE = N_PAD // NS
E_PER_W = N_EDGES // NW
CHUNK = 2000
N_CHUNKS = E_PER_W // CHUNK

ROWS = N_PAD // 128


def _h_body(x_ref, h_ref):
    h_ref[...] = 1.0 / (1.0 + jnp.exp(U - D * x_ref[...]))


def _combine_body(x_ref, p0_ref, p1_ref, f_ref):
    f_ref[...] = p0_ref[...] + p1_ref[...] - x_ref[...]


def _edge_body(h_hbm, src_hbm, dst_hbm, w_hbm, out_hbm,
               h_v, src_v, dst_v, w_v, val_v, z_v, agg_sh):
    c = lax.axis_index("c")
    s = lax.axis_index("s")
    wid = c * NS + s

    def zero_loop(i, _):
        z_v[pl.ds(i * LANES, LANES)] = jnp.zeros((LANES,), jnp.float32)
        return 0
    lax.fori_loop(0, SLICE // LANES, zero_loop, 0)
    pltpu.sync_copy(z_v, agg_sh.at[pl.ds(s * SLICE, SLICE)])

    pltpu.sync_copy(h_hbm, h_v)
    plsc.subcore_barrier()

    base = wid * E_PER_W

    def chunk_loop(j, _):
        off = base + j * CHUNK
        pltpu.sync_copy(src_hbm.at[pl.ds(off, CHUNK)], src_v)
        pltpu.sync_copy(dst_hbm.at[pl.ds(off, CHUNK)], dst_v)
        pltpu.sync_copy(w_hbm.at[pl.ds(off, CHUNK)], w_v)

        def gather_loop(i, _):
            sl = pl.ds(i * LANES, LANES)
            vals = plsc.load_gather(h_v, [src_v[sl]])
            val_v[sl] = vals * w_v[sl]
            return 0
        lax.fori_loop(0, CHUNK // LANES, gather_loop, 0)

        pltpu.sync_copy(val_v, agg_sh.at[dst_v], add=True)
        return 0
    lax.fori_loop(0, N_CHUNKS, chunk_loop, 0)

    plsc.subcore_barrier()
    pltpu.sync_copy(agg_sh.at[pl.ds(s * SLICE, SLICE)],
                    out_hbm.at[pl.ds((c * NS + s) * SLICE, SLICE)])


_edge_kernel = functools.partial(
    pl.kernel,
    out_type=jax.ShapeDtypeStruct((NC * N_PAD,), jnp.float32),
    mesh=plsc.VectorSubcoreMesh(core_axis_name="c", subcore_axis_name="s"),
    compiler_params=pltpu.CompilerParams(needs_layout_passes=False),
    scratch_types=[
        pltpu.VMEM((N_PAD,), jnp.float32),
        pltpu.VMEM((CHUNK,), jnp.int32),
        pltpu.VMEM((CHUNK,), jnp.int32),
        pltpu.VMEM((CHUNK,), jnp.float32),
        pltpu.VMEM((CHUNK,), jnp.float32),
        pltpu.VMEM((SLICE,), jnp.float32),
        pltpu.VMEM_SHARED((N_PAD,), jnp.float32),
    ],
)(_edge_body)


@jax.jit
def kernel(t, x, edge_index, edge_weight):
    del t
    xf = x.reshape(-1)
    x_pad = jnp.pad(xf, (0, N_PAD - N_NODES)).reshape(ROWS, 128)

    h2d = pl.pallas_call(
        _h_body,
        out_shape=jax.ShapeDtypeStruct((ROWS, 128), jnp.float32),
    )(x_pad)
    h = h2d.reshape(N_PAD)

    src = edge_index[0].astype(jnp.int32)
    dst = edge_index[1].astype(jnp.int32)

    partials = _edge_kernel(h, src, dst, edge_weight)
    p0 = partials[:N_PAD].reshape(ROWS, 128)
    p1 = partials[N_PAD:].reshape(ROWS, 128)

    f2d = pl.pallas_call(
        _combine_body,
        out_shape=jax.ShapeDtypeStruct((ROWS, 128), jnp.float32),
    )(x_pad, p0, p1)
    return f2d.reshape(N_PAD)[:N_NODES].reshape(N_NODES, 1)


# speedup vs baseline: 135.7680x; 1.9134x over previous
import functools

import jax
import jax.numpy as jnp
from jax import lax
from jax.experimental import pallas as pl
from jax.experimental.pallas import tpu as pltpu
from jax.experimental.pallas import tpu_sc as plsc

N_NODES = 100000
N_EDGES = 1600000
U = 3.5
D = 2.0

LANES = 16
NC = 2
NS = 16
NW = NC * NS

N_PAD = 100352
SLICE = N_PAD // NS
E_PER_W = N_EDGES // NW
CHUNK = 2000
N_CHUNKS = E_PER_W // CHUNK
NBUF = 3

ROWS = N_PAD // 128


def _h_body(x_ref, h_ref):
    h_ref[...] = 1.0 / (1.0 + jnp.exp(U - D * x_ref[...]))


def _combine_body(x_ref, p_ref, f_ref):
    f_ref[...] = p_ref[:ROWS, :] + p_ref[ROWS:, :] - x_ref[...]


def _edge_body(h_hbm, ei_hbm, w_hbm, z_hbm, out_hbm,
               h_v, src_v, dst_v, w_v, val_v, agg_sh,
               sem_h, sem_z, sem_in, sem_sc):
    c = lax.axis_index("c")
    s = lax.axis_index("s")
    wid = c * NS + s
    base = wid * E_PER_W
    sl = pl.ds(s * SLICE, SLICE)

    zd = pltpu.async_copy(z_hbm.at[sl], agg_sh.at[sl], sem_z)
    hd = pltpu.async_copy(h_hbm.at[pl.ds(0, N_NODES)], h_v, sem_h)

    def issue_inputs(j):
        b = j % NBUF
        off = base + j * CHUNK
        return (
            pltpu.async_copy(ei_hbm.at[pl.ds(off, CHUNK)], src_v[b], sem_in[b]),
            pltpu.async_copy(ei_hbm.at[pl.ds(N_EDGES + off, CHUNK)], dst_v[b], sem_in[b]),
            pltpu.async_copy(w_hbm.at[pl.ds(off, CHUNK)], w_v[b], sem_in[b]),
        )

    in_descs = {0: issue_inputs(0)}
    sc_descs = {}

    zd.wait()
    hd.wait()
    plsc.subcore_barrier()

    for j in range(N_CHUNKS):
        b = j % NBUF
        if j + 1 < N_CHUNKS:
            if j + 1 >= NBUF:
                sc_descs.pop(j + 1 - NBUF).wait()
            in_descs[j + 1] = issue_inputs(j + 1)
        for d in in_descs.pop(j):
            d.wait()

        def gather_loop(i, _):
            ds16 = pl.ds(i * LANES, LANES)
            vals = plsc.load_gather(h_v, [src_v[b][ds16]])
            val_v[b][ds16] = vals * w_v[b][ds16]
            return 0
        lax.fori_loop(0, CHUNK // LANES, gather_loop, 0, unroll=4)

        sc_descs[j] = pltpu.async_copy(
            val_v[b], agg_sh.at[dst_v[b]], sem_sc[b], add=True)

    for j in sorted(sc_descs):
        sc_descs.pop(j).wait()
    plsc.subcore_barrier()
    pltpu.sync_copy(agg_sh.at[sl], out_hbm.at[pl.ds(wid * SLICE, SLICE)])


_edge_kernel = functools.partial(
    pl.kernel,
    out_type=jax.ShapeDtypeStruct((NC * N_PAD,), jnp.float32),
    mesh=plsc.VectorSubcoreMesh(core_axis_name="c", subcore_axis_name="s"),
    compiler_params=pltpu.CompilerParams(needs_layout_passes=False),
    scratch_types=[
        pltpu.VMEM((N_NODES,), jnp.float32),
        [pltpu.VMEM((CHUNK,), jnp.int32)] * NBUF,
        [pltpu.VMEM((CHUNK,), jnp.int32)] * NBUF,
        [pltpu.VMEM((CHUNK,), jnp.float32)] * NBUF,
        [pltpu.VMEM((CHUNK,), jnp.float32)] * NBUF,
        pltpu.VMEM_SHARED((N_PAD,), jnp.float32),
        pltpu.SemaphoreType.DMA,
        pltpu.SemaphoreType.DMA,
        [pltpu.SemaphoreType.DMA] * NBUF,
        [pltpu.SemaphoreType.DMA] * NBUF,
    ],
)(_edge_body)


@jax.jit
def kernel(t, x, edge_index, edge_weight):
    del t
    x_pad = jnp.pad(x, ((0, N_PAD - N_NODES), (0, 0))).reshape(ROWS, 128)

    h2d = pl.pallas_call(
        _h_body,
        out_shape=jax.ShapeDtypeStruct((ROWS, 128), jnp.float32),
    )(x_pad)
    h = h2d.reshape(N_PAD)

    zeros = jnp.zeros((N_PAD,), jnp.float32)
    partials = _edge_kernel(h, edge_index.astype(jnp.int32).reshape(-1), edge_weight, zeros)

    f2d = pl.pallas_call(
        _combine_body,
        out_shape=jax.ShapeDtypeStruct((ROWS, 128), jnp.float32),
    )(x_pad, partials.reshape(2 * ROWS, 128))
    return f2d.reshape(N_PAD)[:N_NODES].reshape(N_NODES, 1)
